# baseline (speedup 1.0000x reference)
import jax
import jax.numpy as jnp
from jax import lax
from jax.experimental import pallas as pl
from jax.experimental.pallas import tpu as pltpu
from jax.experimental.pallas import tpu_sc as plsc

NC = 2
NS = 16
K = 80
KP = (K + 31) // 32 * 16


def _unpack_chunk(buf, j, stage):
    vecs = []
    for blk in range(KP // 16):
        word = buf[j, pl.ds(blk * 16, 16)]
        lo = word & 0xFFFF
        hi = word >> 16
        stage[pl.ds(blk * 32, 16)] = lo
        stage[pl.ds(blk * 32 + 16, 16)] = hi
        vecs += [lo, hi]
    return vecs[:K // 16]


def _sc_pass(table, gidx, sidx, *, n, w, counts, nphase=1,
             edge_split=False):
    R = gidx.shape[0] // NC
    C = R // NS
    CP = C // nphase
    npt = n // NS
    KS = 2 * KP
    NB = 3
    assert C % nphase == 0

    def body(*refs):
        if counts:
            (table_r, gidx_r, sidx_r, out_r, hout_r, gbuf, sbuf,
             *rest) = refs
        else:
            (table_r, gidx_r, sidx_r, out_r, gbuf, sbuf, *rest) = refs
        rows = rest[0:NB]
        gi = rest[NB:2 * NB]
        si = rest[2 * NB:3 * NB]
        acc = rest[3 * NB]
        p = 3 * NB + 1
        if counts:
            hist = rest[p]
            p += 1
        gsem = rest[p:p + NB]
        ssem = rest[p + NB:p + 2 * NB]
        c = lax.axis_index("c")
        s = lax.axis_index("s")

        zv = jnp.zeros((16,), jnp.float32)

        @pl.loop(0, K)
        def _(r):
            for q in range(w // 16):
                rows[0][r, pl.ds(q * 16, 16)] = zv

        base = s * npt
        for off in range(0, npt - K + 1, K):
            pltpu.sync_copy(rows[0], acc.at[pl.ds(base + off, K)])
        tail = npt % K
        if tail:
            pltpu.sync_copy(rows[0].at[pl.ds(0, tail)],
                            acc.at[pl.ds(base + npt - tail, tail)])
        if counts:
            @pl.loop(0, n // 16)
            def _(r):
                hist[pl.ds(r * 16, 16)] = zv
        plsc.subcore_barrier()

        def slot(j, b):
            pltpu.make_async_copy(table_r.at[gi[b].at[pl.ds(0, K)]],
                                  rows[b], gsem[b]).wait()
            svecs = _unpack_chunk(sbuf, j, si[b])
            pltpu.async_copy(rows[b], acc.at[si[b].at[pl.ds(0, K)]],
                             ssem[b], add=True)
            if counts:
                ones = jnp.ones((16,), jnp.float32)

                def _hist():
                    for a in svecs:
                        plsc.addupdate_scatter(hist, [a], ones)

                if counts == 2:
                    _hist()
                else:
                    pl.when(c == 0)(_hist)
            bp = (b + 2) % NB

            @pl.when(j + 2 >= NB)
            def _drain():
                pltpu.make_async_copy(
                    rows[bp], acc.at[si[bp].at[pl.ds(0, K)]],
                    ssem[bp]).wait()

            @pl.when(j + 2 < CP)
            def _issue():
                _unpack_chunk(gbuf, j + 2, gi[bp])
                pltpu.async_copy(table_r.at[gi[bp].at[pl.ds(0, K)]],
                                 rows[bp], gsem[bp])

        CB = CP - CP % NB
        sbase = c * R + s * C if edge_split else s * C
        for ph in range(nphase):
            pltpu.sync_copy(
                gidx_r.at[pl.ds(c * R + s * C + ph * CP, CP)], gbuf)
            pltpu.sync_copy(sidx_r.at[pl.ds(sbase + ph * CP, CP)], sbuf)
            for b in range(2):
                _unpack_chunk(gbuf, b, gi[b])
                pltpu.async_copy(table_r.at[gi[b].at[pl.ds(0, K)]],
                                 rows[b], gsem[b])

            @pl.loop(0, CB, step=NB)
            def _(i):
                for off in range(NB):
                    slot(i + off, off)

            for j in range(CB, CP):
                slot(jnp.int32(j), j % NB)
            for j in range(CP + 2 - NB, CP):
                bl = j % NB
                pltpu.make_async_copy(rows[bl],
                                      acc.at[si[bl].at[pl.ds(0, K)]],
                                      ssem[bl]).wait()

        plsc.subcore_barrier()
        pltpu.sync_copy(acc.at[pl.ds(s * npt, npt)],
                        out_r.at[pl.ds(c * n + s * npt, npt)])
        if counts == 2:
            pltpu.sync_copy(hist, hout_r.at[c * NS + s])
        elif counts == 1:
            @pl.when(c == 0)
            def _hw():
                pltpu.sync_copy(hist, hout_r.at[s])

    out_type = [jax.ShapeDtypeStruct((2 * n, w), jnp.float32)]
    scratch = (
        [pltpu.VMEM((CP, KP), jnp.int32),
         pltpu.VMEM((CP, KP), jnp.int32)]
        + [pltpu.VMEM((K, w), jnp.float32)] * NB
        + [pltpu.VMEM((KS,), jnp.int32)] * (2 * NB)
        + [pltpu.VMEM_SHARED((n, w), jnp.float32)]
    )
    if counts:
        out_type.append(
            jax.ShapeDtypeStruct((NS * counts, n), jnp.float32))
        scratch.append(pltpu.VMEM((n,), jnp.float32))
    scratch += [pltpu.SemaphoreType.DMA] * (2 * NB)

    f = pl.kernel(
        body,
        out_type=tuple(out_type),
        mesh=plsc.VectorSubcoreMesh(core_axis_name="c", subcore_axis_name="s"),
        scratch_types=scratch,
        compiler_params=pltpu.CompilerParams(use_tc_tiling_on_sc=False,
                                             needs_layout_passes=False),
    )
    return f(table, gidx, sidx)


def _pack_idx(idx2d):
    rows = idx2d.shape[0]
    padded = jnp.concatenate(
        [idx2d, jnp.zeros((rows, 16), jnp.int32)], axis=1)
    quads = padded.reshape(rows, KP // 16, 2, 16)
    return (quads[:, :, 0, :] | (quads[:, :, 1, :] << 16)).reshape(rows, KP)


def _tc_dense_body(sum_ref, hist_ref, x_ref, wl_ref, wr_ref, b_ref, g_ref):
    agg = sum_ref[0] + sum_ref[1]
    cnt = jnp.sum(hist_ref[...], axis=0)[:, None]
    mean = agg / jnp.maximum(cnt, 1.0)
    h = jnp.dot(mean, wl_ref[:], preferred_element_type=jnp.float32)
    h += jnp.dot(x_ref[:], wr_ref[:], preferred_element_type=jnp.float32)
    h = jnp.maximum(h + b_ref[:], 0.0)
    g_ref[0] = h
    g_ref[1] = h * h


def _tc_final_body(acc_ref, hist_ref, g_ref, gg_ref):
    s1 = acc_ref[0]
    s2 = acc_ref[1]
    scnt = jnp.sum(hist_ref[...], axis=0)[:, None]
    h = g_ref[0]
    h2 = g_ref[1]
    num = scnt * h2 - 2.0 * h * s1 + s2
    gg_ref[:] = jnp.tanh(num / jnp.maximum(scnt, 1.0))


def kernel(X, edge_index, W_l, W_r, b):
    N, D = X.shape
    E = edge_index.shape[1]
    assert D == 128 and E % (K * NC * NS) == 0 and N % NS == 0

    src = edge_index[0]
    dst = edge_index[1]
    src2d = src.reshape(E // K, K)
    dst2d = dst.reshape(E // K, K)
    allp = _pack_idx(jnp.concatenate(
        [src2d, dst2d, dst2d, dst2d + N], axis=0))
    src2dp = allp[:E // K]
    dst2dp = allp[E // K:2 * (E // K)]
    dstx2d = allp[2 * (E // K):]

    sums, hists_d = _sc_pass(X, src2dp, dst2dp, n=N, w=D, counts=2,
                             nphase=5, edge_split=True)

    g = pl.pallas_call(
        _tc_dense_body,
        out_shape=jax.ShapeDtypeStruct((2, N, D), jnp.float32),
    )(sums.reshape(2, N, D), hists_d, X, W_l, W_r, b.reshape(1, D))

    acc3, hists_s = _sc_pass(g.reshape(2 * N, D), dstx2d, src2dp,
                             n=N, w=D, counts=1, nphase=5)

    gg = pl.pallas_call(
        _tc_final_body,
        out_shape=jax.ShapeDtypeStruct((N, D), jnp.float32),
    )(acc3.reshape(2, N, D), hists_s, g)
    return gg


# speedup vs baseline: 1.0786x; 1.0786x over previous
import jax
import jax.numpy as jnp
from jax import lax
from jax.experimental import pallas as pl
from jax.experimental.pallas import tpu as pltpu
from jax.experimental.pallas import tpu_sc as plsc

NC = 2
NS = 16
K = 80


def _sc_pass(table, gidx, sidx, *, n, w, counts, nphase=1,
             edge_split=False):
    R = gidx.shape[0] // NC
    C = R // NS
    CP = C // nphase
    npt = n // NS
    NB = 3
    assert C % nphase == 0

    def body(*refs):
        if counts:
            (table_r, gidx_r, sidx_r, out_r, hout_r, gbuf, sbuf,
             *rest) = refs
        else:
            (table_r, gidx_r, sidx_r, out_r, gbuf, sbuf, *rest) = refs
        rows = rest[0:NB]
        acc = rest[NB]
        p = NB + 1
        if counts:
            hist = rest[p]
            p += 1
        gsem = rest[p:p + NB]
        ssem = rest[p + NB:p + 2 * NB]
        c = lax.axis_index("c")
        s = lax.axis_index("s")

        zv = jnp.zeros((16,), jnp.float32)

        @pl.loop(0, K)
        def _(r):
            for q in range(w // 16):
                rows[0][r, pl.ds(q * 16, 16)] = zv

        base = s * npt
        for off in range(0, npt - K + 1, K):
            pltpu.sync_copy(rows[0], acc.at[pl.ds(base + off, K)])
        tail = npt % K
        if tail:
            pltpu.sync_copy(rows[0].at[pl.ds(0, tail)],
                            acc.at[pl.ds(base + npt - tail, tail)])
        if counts:
            @pl.loop(0, n // 16)
            def _(r):
                hist[pl.ds(r * 16, 16)] = zv
        plsc.subcore_barrier()

        def slot(j, b):
            pltpu.make_async_copy(table_r.at[gbuf.at[j]],
                                  rows[b], gsem[b]).wait()
            pltpu.async_copy(rows[b], acc.at[sbuf.at[j]],
                             ssem[b], add=True)
            if counts:
                ones = jnp.ones((16,), jnp.float32)

                def _hist():
                    for t in range(K // 16):
                        a = sbuf[j, pl.ds(t * 16, 16)]
                        plsc.addupdate_scatter(hist, [a], ones)

                if counts == 2:
                    _hist()
                else:
                    pl.when(c == 0)(_hist)
            bp = (b + 2) % NB

            @pl.when(j + 2 >= NB)
            def _drain():
                pltpu.make_async_copy(
                    rows[bp], acc.at[sbuf.at[j - 1]],
                    ssem[bp]).wait()

            @pl.when(j + 2 < CP)
            def _issue():
                pltpu.async_copy(table_r.at[gbuf.at[j + 2]],
                                 rows[bp], gsem[bp])

        CB = CP - CP % NB
        sbase = c * R + s * C if edge_split else s * C
        for ph in range(nphase):
            pltpu.sync_copy(
                gidx_r.at[pl.ds(c * R + s * C + ph * CP, CP)], gbuf)
            pltpu.sync_copy(sidx_r.at[pl.ds(sbase + ph * CP, CP)], sbuf)
            for b in range(2):
                pltpu.async_copy(table_r.at[gbuf.at[b]], rows[b], gsem[b])

            @pl.loop(0, CB, step=NB)
            def _(i):
                for off in range(NB):
                    slot(i + off, off)

            for j in range(CB, CP):
                slot(jnp.int32(j), j % NB)
            for j in range(CP + 2 - NB, CP):
                pltpu.make_async_copy(rows[j % NB], acc.at[sbuf.at[j]],
                                      ssem[j % NB]).wait()

        plsc.subcore_barrier()
        pltpu.sync_copy(acc.at[pl.ds(s * npt, npt)],
                        out_r.at[pl.ds(c * n + s * npt, npt)])
        if counts == 2:
            pltpu.sync_copy(hist, hout_r.at[c * NS + s])
        elif counts == 1:
            @pl.when(c == 0)
            def _hw():
                pltpu.sync_copy(hist, hout_r.at[s])

    out_type = [jax.ShapeDtypeStruct((2 * n, w), jnp.float32)]
    scratch = (
        [pltpu.VMEM((CP, K), jnp.int32),
         pltpu.VMEM((CP, K), jnp.int32)]
        + [pltpu.VMEM((K, w), jnp.float32)] * NB
        + [pltpu.VMEM_SHARED((n, w), jnp.float32)]
    )
    if counts:
        out_type.append(
            jax.ShapeDtypeStruct((NS * counts, n), jnp.float32))
        scratch.append(pltpu.VMEM((n,), jnp.float32))
    scratch += [pltpu.SemaphoreType.DMA] * (2 * NB)

    f = pl.kernel(
        body,
        out_type=tuple(out_type),
        mesh=plsc.VectorSubcoreMesh(core_axis_name="c", subcore_axis_name="s"),
        scratch_types=scratch,
        compiler_params=pltpu.CompilerParams(use_tc_tiling_on_sc=False,
                                             needs_layout_passes=False),
    )
    return f(table, gidx, sidx)


def _tc_dense_body(sum_ref, hist_ref, x_ref, wl_ref, wr_ref, b_ref, g_ref):
    agg = sum_ref[0] + sum_ref[1]
    cnt = jnp.sum(hist_ref[...], axis=0)[:, None]
    mean = agg / jnp.maximum(cnt, 1.0)
    h = jnp.dot(mean, wl_ref[:], preferred_element_type=jnp.float32)
    h += jnp.dot(x_ref[:], wr_ref[:], preferred_element_type=jnp.float32)
    h = jnp.maximum(h + b_ref[:], 0.0)
    g_ref[0] = h
    g_ref[1] = h * h


def _tc_final_body(acc_ref, hist_ref, g_ref, gg_ref):
    s1 = acc_ref[0]
    s2 = acc_ref[1]
    scnt = jnp.sum(hist_ref[...], axis=0)[:, None]
    h = g_ref[0]
    h2 = g_ref[1]
    num = scnt * h2 - 2.0 * h * s1 + s2
    gg_ref[:] = jnp.tanh(num / jnp.maximum(scnt, 1.0))


def kernel(X, edge_index, W_l, W_r, b):
    N, D = X.shape
    E = edge_index.shape[1]
    assert D == 128 and E % (K * NC * NS) == 0 and N % NS == 0

    src = edge_index[0]
    dst = edge_index[1]
    src2d = src.reshape(E // K, K)
    dst2d = dst.reshape(E // K, K)
    dstx2d = jnp.concatenate([dst2d, dst2d + N], axis=0)
    src2dp = src2d
    dst2dp = dst2d

    sums, hists_d = _sc_pass(X, src2dp, dst2dp, n=N, w=D, counts=2,
                             nphase=5, edge_split=True)

    g = pl.pallas_call(
        _tc_dense_body,
        out_shape=jax.ShapeDtypeStruct((2, N, D), jnp.float32),
    )(sums.reshape(2, N, D), hists_d, X, W_l, W_r, b.reshape(1, D))

    acc3, hists_s = _sc_pass(g.reshape(2 * N, D), dstx2d, src2dp,
                             n=N, w=D, counts=1, nphase=5)

    gg = pl.pallas_call(
        _tc_final_body,
        out_shape=jax.ShapeDtypeStruct((N, D), jnp.float32),
    )(acc3.reshape(2, N, D), hists_s, g)
    return gg
